# baseline (speedup 1.0000x reference)
import jax
import jax.numpy as jnp
from jax import lax
from jax.experimental import pallas as pl
from jax.experimental.pallas import tpu as pltpu
from jax.experimental.pallas import tpu_sc as plsc

NC = 2
NS = 16
L = 16
NW = NC * NS
B = 16384
D = 32
BPW = B // NW
GROUPS = BPW // L
CHUNK = 128


def _dot_body(users_hbm, items_hbm, uemb_hbm, iemb_hbm, out_hbm,
              uidx_v, iidx_v, urows_v, irows_v, hbuf_v, out_v, sem_u, sem_i):
    wid = lax.axis_index("s") * NC + lax.axis_index("c")
    base = wid * BPW

    pltpu.sync_copy(users_hbm.at[pl.ds(base, BPW)], uidx_v)
    pltpu.sync_copy(items_hbm.at[pl.ds(base, BPW)], iidx_v)

    copies = []
    for k in range(BPW // CHUNK):
        sl = pl.ds(k * CHUNK, CHUNK)
        copies.append(pltpu.async_copy(
            uemb_hbm.at[uidx_v.at[sl]], urows_v.at[sl], sem_u))
        copies.append(pltpu.async_copy(
            iemb_hbm.at[iidx_v.at[sl]], irows_v.at[sl], sem_i))
    for cp in copies:
        cp.wait()

    lanes = lax.iota(jnp.int32, L)
    scatter_base = lanes * BPW

    def row_pass(r0, carry):
        for k in range(4):
            r = r0 * 4 + k
            u0 = urows_v[r, pl.ds(0, L)]
            u1 = urows_v[r, pl.ds(L, L)]
            i0 = irows_v[r, pl.ds(0, L)]
            i1 = irows_v[r, pl.ds(L, L)]
            h = u0 * i0 + u1 * i1
            plsc.store_scatter(hbuf_v, [scatter_base + r], h)
        return carry

    lax.fori_loop(0, BPW // 4, row_pass, 0)

    def group_pass(g, carry):
        acc = jnp.zeros((L,), jnp.float32)
        for d in range(L):
            acc = acc + hbuf_v[pl.ds(d * BPW + g * L, L)]
        out_v[pl.ds(g * L, L)] = acc
        return carry

    lax.fori_loop(0, GROUPS, group_pass, 0)

    pltpu.sync_copy(out_v, out_hbm.at[pl.ds(base, BPW)])


def kernel(users, items, user_emb, item_emb):
    mesh = plsc.VectorSubcoreMesh(core_axis_name="c", subcore_axis_name="s")
    run = pl.kernel(
        _dot_body,
        out_type=jax.ShapeDtypeStruct((B,), jnp.float32),
        mesh=mesh,
        compiler_params=pltpu.CompilerParams(
            needs_layout_passes=False, use_tc_tiling_on_sc=False),
        scratch_types=[
            pltpu.VMEM((BPW,), jnp.int32),
            pltpu.VMEM((BPW,), jnp.int32),
            pltpu.VMEM((BPW, D), jnp.float32),
            pltpu.VMEM((BPW, D), jnp.float32),
            pltpu.VMEM((L * BPW,), jnp.float32),
            pltpu.VMEM((BPW,), jnp.float32),
            pltpu.SemaphoreType.DMA,
            pltpu.SemaphoreType.DMA,
        ],
    )
    return run(users.astype(jnp.int32), items.astype(jnp.int32),
               user_emb, item_emb)


# speedup vs baseline: 1.4911x; 1.4911x over previous
import jax
import jax.numpy as jnp
from jax import lax
from jax.experimental import pallas as pl
from jax.experimental.pallas import tpu as pltpu
from jax.experimental.pallas import tpu_sc as plsc

NC = 2
NS = 16
L = 16
NW = NC * NS
B = 16384
D = 32
BPW = B // NW
CR = 128
NCH = BPW // CR
CGROUPS = CR // L


def _dot_body(users_hbm, items_hbm, uemb_hbm, iemb_hbm, out_hbm,
              uidx_v, iidx_v, ubuf_v, ibuf_v, hbuf_v, out_v, sem_u, sem_i):
    wid = lax.axis_index("s") * NC + lax.axis_index("c")
    base = wid * BPW

    pltpu.sync_copy(users_hbm.at[pl.ds(base, BPW)], uidx_v)
    pltpu.sync_copy(items_hbm.at[pl.ds(base, BPW)], iidx_v)

    lanes = lax.iota(jnp.int32, L)
    scatter_base = lanes * CR

    def chunk(c, carry):
        def enq(j0, carry2):
            uvec = uidx_v[pl.ds(c * CR + j0 * L, L)]
            ivec = iidx_v[pl.ds(c * CR + j0 * L, L)]
            for k in range(L):
                j = j0 * L + k
                pltpu.async_copy(uemb_hbm.at[pl.ds(uvec[k], 1), :],
                                 ubuf_v.at[pl.ds(j, 1), :], sem_u)
                pltpu.async_copy(iemb_hbm.at[pl.ds(ivec[k], 1), :],
                                 ibuf_v.at[pl.ds(j, 1), :], sem_i)
            return carry2

        lax.fori_loop(0, CR // L, enq, 0)
        pltpu.make_async_copy(uemb_hbm.at[pl.ds(0, CR), :], ubuf_v,
                              sem_u).wait()
        pltpu.make_async_copy(iemb_hbm.at[pl.ds(0, CR), :], ibuf_v,
                              sem_i).wait()

        def row_pass(r0, carry2):
            for k in range(4):
                r = r0 * 4 + k
                u0 = ubuf_v[r, pl.ds(0, L)]
                u1 = ubuf_v[r, pl.ds(L, L)]
                i0 = ibuf_v[r, pl.ds(0, L)]
                i1 = ibuf_v[r, pl.ds(L, L)]
                h = u0 * i0 + u1 * i1
                plsc.store_scatter(hbuf_v, [scatter_base + r], h)
            return carry2

        lax.fori_loop(0, CR // 4, row_pass, 0)

        def group_pass(g, carry2):
            acc = jnp.zeros((L,), jnp.float32)
            for d in range(L):
                acc = acc + hbuf_v[pl.ds(d * CR + g * L, L)]
            out_v[pl.ds(c * CR + g * L, L)] = acc
            return carry2

        lax.fori_loop(0, CGROUPS, group_pass, 0)
        return carry

    lax.fori_loop(0, NCH, chunk, 0)

    pltpu.sync_copy(out_v, out_hbm.at[pl.ds(base, BPW)])


def kernel(users, items, user_emb, item_emb):
    mesh = plsc.VectorSubcoreMesh(core_axis_name="c", subcore_axis_name="s")
    run = pl.kernel(
        _dot_body,
        out_type=jax.ShapeDtypeStruct((B,), jnp.float32),
        mesh=mesh,
        compiler_params=pltpu.CompilerParams(
            needs_layout_passes=False, use_tc_tiling_on_sc=True),
        scratch_types=[
            pltpu.VMEM((BPW,), jnp.int32),
            pltpu.VMEM((BPW,), jnp.int32),
            pltpu.VMEM((CR, D), jnp.float32),
            pltpu.VMEM((CR, D), jnp.float32),
            pltpu.VMEM((L * CR,), jnp.float32),
            pltpu.VMEM((BPW,), jnp.float32),
            pltpu.SemaphoreType.DMA,
            pltpu.SemaphoreType.DMA,
        ],
    )
    return run(users.astype(jnp.int32), items.astype(jnp.int32),
               user_emb, item_emb)
